# baseline (speedup 1.0000x reference)
import math

import jax
import jax.numpy as jnp
import numpy as np
from jax import lax
from jax.experimental import pallas as pl
from jax.experimental.pallas import tpu as pltpu
from jax.experimental.pallas import tpu_sc as plsc

BLOCK_SIZE = 200
DMODEL = 32
BATCH = 4096
SCALE = math.sqrt(DMODEL)

NUM_CORES = 2
NUM_SUBCORES = 16
NUM_WORKERS = NUM_CORES * NUM_SUBCORES
BPW = BATCH // NUM_WORKERS
TCHUNK = 4
ROWS = TCHUNK * BPW
NCHUNKS = BLOCK_SIZE // TCHUNK
NBUF = 2
NGROUPS = NCHUNKS // NBUF


def _positional_embeddings_scaled():
    pe = np.zeros((BLOCK_SIZE, DMODEL), dtype=np.float32)
    pos = np.arange(BLOCK_SIZE, dtype=np.float32)[:, None]
    i = np.arange(DMODEL // 2, dtype=np.float32)[None, :]
    denom = np.power(10000.0, 2.0 * i / DMODEL)
    pe[:, 0::2] = np.sin(pos / denom)
    pe[:, 1::2] = np.cos(pos / denom)
    return jnp.asarray(pe * SCALE)


_MESH = plsc.VectorSubcoreMesh(
    core_axis_name="c", subcore_axis_name="s",
    num_cores=NUM_CORES, num_subcores=NUM_SUBCORES,
)


@jax.jit
def _embed(x, w, pe_s):
    x_t = x.T

    @pl.kernel(
        out_type=jax.ShapeDtypeStruct((BLOCK_SIZE, DMODEL, BATCH), jnp.float32),
        mesh=_MESH,
        scratch_types=(
            [pltpu.VMEM((BLOCK_SIZE, BPW), jnp.int32)]
            + [pltpu.VMEM((ROWS, DMODEL), jnp.float32) for _ in range(NBUF)]
            + [pltpu.VMEM((TCHUNK, DMODEL, BPW), jnp.float32) for _ in range(NBUF)]
            + [pltpu.VMEM((BLOCK_SIZE, DMODEL), jnp.float32)]
            + [pltpu.SemaphoreType.DMA for _ in range(2 * NBUF)]
        ),
        compiler_params=pltpu.CompilerParams(
            use_tc_tiling_on_sc=False, needs_layout_passes=False),
    )
    def body(xt_hbm, w_hbm, pe_hbm, out_hbm, idx_t, *rest):
        gbufs = rest[:NBUF]
        wbufs = rest[NBUF:2 * NBUF]
        pe_v = rest[2 * NBUF]
        gsems = rest[2 * NBUF + 1:2 * NBUF + 1 + NBUF]
        wsems = rest[2 * NBUF + 1 + NBUF:]

        wid = lax.axis_index("s") * NUM_CORES + lax.axis_index("c")
        b0 = wid * BPW
        pltpu.sync_copy(pe_hbm, pe_v)
        pltpu.sync_copy(
            xt_hbm.at[pl.ds(0, BLOCK_SIZE), pl.ds(b0, BPW)], idx_t)

        def gather(c, s):
            for tl in range(TCHUNK):
                pltpu.async_copy(
                    w_hbm.at[idx_t.at[c * TCHUNK + tl]],
                    gbufs[s].at[pl.ds(tl * BPW, BPW)], gsems[s])

        for s in range(NBUF):
            gather(s, s)

        iota = lax.iota(jnp.int32, 16)
        rowids = [[iota + (tl * BPW + k * 16) for k in range(BPW // 16)]
                  for tl in range(TCHUNK)]

        def group(g, carry):
            for s in range(NBUF):
                c = g * NBUF + s
                pltpu.make_async_copy(
                    w_hbm.at[pl.ds(0, ROWS)], gbufs[s], gsems[s]).wait()
                @pl.when(g > 0)
                def _():
                    pltpu.make_async_copy(
                        wbufs[s],
                        out_hbm.at[pl.ds(0, TCHUNK), pl.ds(0, DMODEL),
                                   pl.ds(0, BPW)],
                        wsems[s]).wait()

                t0 = c * TCHUNK

                def d_body(d, cy):
                    col = jnp.broadcast_to(d, (16,)).astype(jnp.int32)
                    for tl in range(TCHUNK):
                        pe_sc = plsc.load_gather(
                            pe_v, [jnp.broadcast_to(t0 + tl, (16,)), col])
                        for k in range(BPW // 16):
                            val = plsc.load_gather(
                                gbufs[s], [rowids[tl][k], col])
                            wbufs[s][tl, d, pl.ds(k * 16, 16)] = (
                                val * SCALE + pe_sc)
                    return cy

                lax.fori_loop(0, DMODEL, d_body, 0)

                pltpu.async_copy(
                    wbufs[s],
                    out_hbm.at[pl.ds(t0, TCHUNK), pl.ds(0, DMODEL),
                               pl.ds(b0, BPW)],
                    wsems[s])

                @pl.when(c + NBUF < NCHUNKS)
                def _():
                    gather(c + NBUF, s)
            return carry

        lax.fori_loop(0, NGROUPS, group, 0)

        for s in range(NBUF):
            pltpu.make_async_copy(
                wbufs[s],
                out_hbm.at[pl.ds(0, TCHUNK), pl.ds(0, DMODEL), pl.ds(0, BPW)],
                wsems[s]).wait()

    out3 = body(x_t, w, pe_s)
    return jnp.transpose(out3, (2, 0, 1))


def kernel(x, W):
    pe_s = _positional_embeddings_scaled()
    return _embed(x.astype(jnp.int32), W, pe_s)


# speedup vs baseline: 1.4409x; 1.4409x over previous
import math

import jax
import jax.numpy as jnp
import numpy as np
from jax import lax
from jax.experimental import pallas as pl
from jax.experimental.pallas import tpu as pltpu
from jax.experimental.pallas import tpu_sc as plsc

BLOCK_SIZE = 200
DMODEL = 32
BATCH = 4096
SCALE = math.sqrt(DMODEL)

NUM_CORES = 2
NUM_SUBCORES = 16
NUM_WORKERS = NUM_CORES * NUM_SUBCORES
BPW = BATCH // NUM_WORKERS
TCHUNK = 4
ROWS = TCHUNK * BPW
NCHUNKS = BLOCK_SIZE // TCHUNK
NBUF = 2
NGROUPS = NCHUNKS // NBUF


def _positional_embeddings_scaled():
    pe = np.zeros((BLOCK_SIZE, DMODEL), dtype=np.float32)
    pos = np.arange(BLOCK_SIZE, dtype=np.float32)[:, None]
    i = np.arange(DMODEL // 2, dtype=np.float32)[None, :]
    denom = np.power(10000.0, 2.0 * i / DMODEL)
    pe[:, 0::2] = np.sin(pos / denom)
    pe[:, 1::2] = np.cos(pos / denom)
    return jnp.asarray(pe * SCALE)


_MESH = plsc.VectorSubcoreMesh(
    core_axis_name="c", subcore_axis_name="s",
    num_cores=NUM_CORES, num_subcores=NUM_SUBCORES,
)


@jax.jit
def _embed(x, w, pe_s):
    x_t = x.T

    @pl.kernel(
        out_type=jax.ShapeDtypeStruct((BLOCK_SIZE, DMODEL, BATCH), jnp.float32),
        mesh=_MESH,
        scratch_types=(
            [pltpu.VMEM((BLOCK_SIZE, BPW), jnp.int32)]
            + [pltpu.VMEM((ROWS, DMODEL), jnp.float32) for _ in range(NBUF)]
            + [pltpu.VMEM((TCHUNK, DMODEL, BPW + 1), jnp.float32)
               for _ in range(NBUF)]
            + [pltpu.VMEM((BLOCK_SIZE, DMODEL), jnp.float32)]
            + [pltpu.SemaphoreType.DMA for _ in range(2 * NBUF)]
        ),
        compiler_params=pltpu.CompilerParams(
            use_tc_tiling_on_sc=False, needs_layout_passes=False),
    )
    def body(xt_hbm, w_hbm, pe_hbm, out_hbm, idx_t, *rest):
        gbufs = rest[:NBUF]
        wbufs = rest[NBUF:2 * NBUF]
        pe_v = rest[2 * NBUF]
        gsems = rest[2 * NBUF + 1:2 * NBUF + 1 + NBUF]
        wsems = rest[2 * NBUF + 1 + NBUF:]

        wid = lax.axis_index("s") * NUM_CORES + lax.axis_index("c")
        b0 = wid * BPW
        pltpu.sync_copy(pe_hbm, pe_v)
        pltpu.sync_copy(
            xt_hbm.at[pl.ds(0, BLOCK_SIZE), pl.ds(b0, BPW)], idx_t)

        def gather(c, s):
            for tl in range(TCHUNK):
                pltpu.async_copy(
                    w_hbm.at[idx_t.at[c * TCHUNK + tl]],
                    gbufs[s].at[pl.ds(tl * BPW, BPW)], gsems[s])

        for s in range(NBUF):
            gather(s, s)

        iota = lax.iota(jnp.int32, 16)
        dvecs = [iota + h * 16 for h in range(2)]
        tlvecs = [jnp.broadcast_to(jnp.int32(tl), (16,))
                  for tl in range(TCHUNK)]

        def group(g, carry):
            for s in range(NBUF):
                c = g * NBUF + s
                pltpu.make_async_copy(
                    w_hbm.at[pl.ds(0, ROWS)], gbufs[s], gsems[s]).wait()
                @pl.when(g > 0)
                def _():
                    pltpu.make_async_copy(
                        wbufs[s].at[pl.ds(0, TCHUNK), pl.ds(0, DMODEL),
                                    pl.ds(0, BPW)],
                        out_hbm.at[pl.ds(0, TCHUNK), pl.ds(0, DMODEL),
                                   pl.ds(0, BPW)],
                        wsems[s]).wait()

                t0 = c * TCHUNK
                pes = [[pe_v[t0 + tl, pl.ds(h * 16, 16)] for h in range(2)]
                       for tl in range(TCHUNK)]

                def b_body(bb, cy):
                    for bs in range(4):
                        b = bb * 4 + bs
                        bvec = jnp.broadcast_to(b, (16,)).astype(jnp.int32)
                        for tl in range(TCHUNK):
                            row = tl * BPW + b
                            for h in range(2):
                                y = (gbufs[s][row, pl.ds(h * 16, 16)] * SCALE
                                     + pes[tl][h])
                                plsc.store_scatter(
                                    wbufs[s], [tlvecs[tl], dvecs[h], bvec], y)
                    return cy

                lax.fori_loop(0, BPW // 4, b_body, 0)

                pltpu.async_copy(
                    wbufs[s].at[pl.ds(0, TCHUNK), pl.ds(0, DMODEL),
                                pl.ds(0, BPW)],
                    out_hbm.at[pl.ds(t0, TCHUNK), pl.ds(0, DMODEL),
                               pl.ds(b0, BPW)],
                    wsems[s])

                @pl.when(c + NBUF < NCHUNKS)
                def _():
                    gather(c + NBUF, s)
            return carry

        lax.fori_loop(0, NGROUPS, group, 0)

        for s in range(NBUF):
            pltpu.make_async_copy(
                wbufs[s].at[pl.ds(0, TCHUNK), pl.ds(0, DMODEL),
                            pl.ds(0, BPW)],
                out_hbm.at[pl.ds(0, TCHUNK), pl.ds(0, DMODEL), pl.ds(0, BPW)],
                wsems[s]).wait()

    out3 = body(x_t, w, pe_s)
    return jnp.transpose(out3, (2, 0, 1))


def kernel(x, W):
    pe_s = _positional_embeddings_scaled()
    return _embed(x.astype(jnp.int32), W, pe_s)


# speedup vs baseline: 1.6481x; 1.1438x over previous
import math

import jax
import jax.numpy as jnp
import numpy as np
from jax import lax
from jax.experimental import pallas as pl
from jax.experimental.pallas import tpu as pltpu
from jax.experimental.pallas import tpu_sc as plsc

BLOCK_SIZE = 200
DMODEL = 32
VOCAB_ROWS = 1000000
BATCH = 4096
SCALE = math.sqrt(DMODEL)

NUM_CORES = 2
NUM_SUBCORES = 16
NUM_WORKERS = NUM_CORES * NUM_SUBCORES
BPW = BATCH // NUM_WORKERS
TCHUNK = 4
ROWS = TCHUNK * BPW
NCHUNKS = BLOCK_SIZE // TCHUNK
NBUF = 2
NGROUPS = NCHUNKS // NBUF
DTILES = DMODEL // 8


def _positional_embeddings_scaled():
    pe = np.zeros((BLOCK_SIZE, DMODEL), dtype=np.float32)
    pos = np.arange(BLOCK_SIZE, dtype=np.float32)[:, None]
    i = np.arange(DMODEL // 2, dtype=np.float32)[None, :]
    denom = np.power(10000.0, 2.0 * i / DMODEL)
    pe[:, 0::2] = np.sin(pos / denom)
    pe[:, 1::2] = np.cos(pos / denom)
    return jnp.asarray(pe * SCALE)


_MESH = plsc.VectorSubcoreMesh(
    core_axis_name="c", subcore_axis_name="s",
    num_cores=NUM_CORES, num_subcores=NUM_SUBCORES,
)


@jax.jit
def _embed(x, w, pe_s):
    x_t = x.T
    w2 = lax.optimization_barrier(w.reshape(VOCAB_ROWS * DMODEL // 128, 128))
    w_rows = w2.reshape(VOCAB_ROWS, DMODEL)

    @pl.kernel(
        out_type=jax.ShapeDtypeStruct(
            (BLOCK_SIZE, DTILES, NUM_WORKERS, 8, BPW), jnp.float32),
        mesh=_MESH,
        scratch_types=(
            [pltpu.VMEM((BLOCK_SIZE, BPW), jnp.int32)]
            + [pltpu.VMEM((ROWS, DMODEL), jnp.float32) for _ in range(NBUF)]
            + [pltpu.VMEM((TCHUNK, DTILES, 8, BPW + 1), jnp.float32)
               for _ in range(NBUF)]
            + [pltpu.VMEM((BLOCK_SIZE, DMODEL), jnp.float32)]
            + [pltpu.SemaphoreType.DMA for _ in range(2 * NBUF)]
        ),
        compiler_params=pltpu.CompilerParams(
            use_tc_tiling_on_sc=False, needs_layout_passes=False),
    )
    def body(xt_hbm, w_hbm, pe_hbm, out_hbm, idx_t, *rest):
        gbufs = rest[:NBUF]
        wbufs = rest[NBUF:2 * NBUF]
        pe_v = rest[2 * NBUF]
        gsems = rest[2 * NBUF + 1:2 * NBUF + 1 + NBUF]
        wsems = rest[2 * NBUF + 1 + NBUF:]

        wid = lax.axis_index("s") * NUM_CORES + lax.axis_index("c")
        b0 = wid * BPW
        pltpu.sync_copy(pe_hbm, pe_v)
        pltpu.sync_copy(
            xt_hbm.at[pl.ds(0, BLOCK_SIZE), pl.ds(b0, BPW)], idx_t)

        def gather(c, s):
            for tl in range(TCHUNK):
                pltpu.async_copy(
                    w_hbm.at[idx_t.at[c * TCHUNK + tl]],
                    gbufs[s].at[pl.ds(tl * BPW, BPW)], gsems[s])

        for s in range(NBUF):
            gather(s, s)

        iota = lax.iota(jnp.int32, 16)
        tdvecs = [(iota + h * 16) // 8 for h in range(2)]
        sdvecs = [(iota + h * 16) % 8 for h in range(2)]
        tlvecs = [jnp.broadcast_to(jnp.int32(tl), (16,))
                  for tl in range(TCHUNK)]

        def group(g, carry):
            for s in range(NBUF):
                c = g * NBUF + s
                pltpu.make_async_copy(
                    w_hbm.at[pl.ds(0, ROWS)], gbufs[s], gsems[s]).wait()
                @pl.when(g > 0)
                def _():
                    pltpu.make_async_copy(
                        wbufs[s].at[pl.ds(0, TCHUNK), pl.ds(0, DTILES),
                                    pl.ds(0, 8), pl.ds(0, BPW)],
                        out_hbm.at[pl.ds(0, TCHUNK), pl.ds(0, DTILES), 0,
                                   pl.ds(0, 8), pl.ds(0, BPW)],
                        wsems[s]).wait()

                t0 = c * TCHUNK
                pes = [[pe_v[t0 + tl, pl.ds(h * 16, 16)] for h in range(2)]
                       for tl in range(TCHUNK)]

                def b_body(bb, cy):
                    for bs in range(4):
                        b = bb * 4 + bs
                        bvec = jnp.broadcast_to(b, (16,)).astype(jnp.int32)
                        for tl in range(TCHUNK):
                            row = tl * BPW + b
                            for h in range(2):
                                y = (gbufs[s][row, pl.ds(h * 16, 16)] * SCALE
                                     + pes[tl][h])
                                plsc.store_scatter(
                                    wbufs[s],
                                    [tlvecs[tl], tdvecs[h], sdvecs[h], bvec],
                                    y)
                    return cy

                lax.fori_loop(0, BPW // 4, b_body, 0)

                pltpu.async_copy(
                    wbufs[s].at[pl.ds(0, TCHUNK), pl.ds(0, DTILES),
                                pl.ds(0, 8), pl.ds(0, BPW)],
                    out_hbm.at[pl.ds(t0, TCHUNK), pl.ds(0, DTILES), wid,
                               pl.ds(0, 8), pl.ds(0, BPW)],
                    wsems[s])

                @pl.when(c + NBUF < NCHUNKS)
                def _():
                    gather(c + NBUF, s)
            return carry

        lax.fori_loop(0, NGROUPS, group, 0)

        for s in range(NBUF):
            pltpu.make_async_copy(
                wbufs[s].at[pl.ds(0, TCHUNK), pl.ds(0, DTILES),
                            pl.ds(0, 8), pl.ds(0, BPW)],
                out_hbm.at[pl.ds(0, TCHUNK), pl.ds(0, DTILES), 0,
                           pl.ds(0, 8), pl.ds(0, BPW)],
                wsems[s]).wait()

    out6 = body(x_t, w_rows, pe_s)
    return out6.transpose(2, 4, 0, 1, 3).reshape(BATCH, BLOCK_SIZE, DMODEL)


def kernel(x, W):
    pe_s = _positional_embeddings_scaled()
    return _embed(x.astype(jnp.int32), W, pe_s)
